# baseline (speedup 1.0000x reference)
import functools

import jax
import jax.numpy as jnp
from jax import lax
from jax.experimental import pallas as pl
from jax.experimental.pallas import tpu as pltpu
from jax.experimental.pallas import tpu_sc as plsc

_NC = 2
_NS = 16
_LANES = 16


def _mesh():
    return plsc.VectorSubcoreMesh(
        core_axis_name="c", subcore_axis_name="s",
        num_cores=_NC, num_subcores=_NS)


_LPAD = 24
_W = 512


def _build_prep(F, B, L, V, D):
    NW = _NC * _NS
    NBLK = B // 128
    VFULL = (V // 128) * 128
    VTAIL = V - VFULL
    NPAN = VFULL // _W
    KPW = NPAN // NW
    NEXTRA = NPAN - KPW * NW
    assert NEXTRA <= 1 and VTAIL in (0, 64)
    OUTROWS = V * D // 128

    @functools.partial(
        pl.kernel,
        out_type=(
            jax.ShapeDtypeStruct((OUTROWS, 128), jnp.float32),
            jax.ShapeDtypeStruct((F, NBLK, _LPAD, 128), jnp.int32),
        ),
        mesh=_mesh(),
        compiler_params=pltpu.CompilerParams(use_tc_tiling_on_sc=True, needs_layout_passes=False),
        scratch_types=[
            pltpu.VMEM((_LPAD, 128), jnp.int32),
            pltpu.VMEM((_LPAD, 128), jnp.int32),
            pltpu.VMEM((D, _W + 1), jnp.float32),
            pltpu.VMEM((D, _W + 1), jnp.float32),
            pltpu.VMEM((32, 128), jnp.float32),
            pltpu.VMEM((32, 128), jnp.float32),
            pltpu.VMEM((32, 128), jnp.float32),
            pltpu.VMEM((32, 128), jnp.float32),
            pltpu.VMEM((VTAIL, D), jnp.float32),
            pltpu.SemaphoreType.DMA,
            pltpu.SemaphoreType.DMA,
            pltpu.SemaphoreType.DMA,
            pltpu.SemaphoreType.DMA,
            pltpu.SemaphoreType.DMA,
            pltpu.SemaphoreType.DMA,
            pltpu.SemaphoreType.DMA,
        ],
    )
    def prep(tbt_hbm, tail_hbm, idx_hbm, tbl_out, idx_out,
             ist0, ist1, tin0, tin1, tout0, tout1, tout2, tout3, tailst,
             sem_i, semr0, semr1, semw0, semw1, semw2, semw3):
        wid = lax.axis_index("s") * _NC + lax.axis_index("c")

        ists = [ist0, ist1]
        pending = [None, None]
        for f in range(F):
            stage = ists[f % 2]
            if pending[f % 2] is not None:
                pending[f % 2].wait()
                pending[f % 2] = None
            pltpu.sync_copy(
                idx_hbm.at[f, :, pl.ds(pl.multiple_of(wid * 128, 128), 128)],
                stage.at[pl.ds(0, L), :])
            pending[f % 2] = pltpu.async_copy(
                stage, idx_out.at[f, wid], [sem_i, semr0][f % 2])
        for cp in pending:
            if cp is not None:
                cp.wait()

        iota = lax.iota(jnp.int32, _LANES)
        jv = [iota, iota + _LANES]
        tins = [tin0, tin1]
        semrs = [semr0, semr1]
        touts = [tout0, tout1, tout2, tout3]
        semws = [semw0, semw1, semw2, semw3]
        p_extra = jnp.int32(NPAN - 1)

        def p_of(k):
            base = wid * KPW + k
            if NEXTRA:
                return jnp.where(k == KPW, p_extra, base)
            return base

        def fire_read(k, buf, sem):
            c0 = pl.multiple_of(p_of(k) * _W, _W)
            return pltpu.async_copy(tbt_hbm.at[:, pl.ds(c0, _W)],
                                buf.at[:, pl.ds(0, _W)], sem)

        def drain_write(b):
            pltpu.make_async_copy(tbl_out.at[pl.ds(0, 32), :], touts[b],
                                  semws[b]).wait()

        def compute_panel(k, buf, not_first):
            p = p_of(k)
            row0 = p * 128

            for s in range(4):
                @pl.when(not_first)
                def _(s=s):
                    drain_write(s)
                rb = jnp.zeros((_LANES,), jnp.int32) + s * 128
                out = touts[s]

                @plsc.parallel_loop(0, 16, step=1, unroll=4)
                def grp(g, s=s, rb=rb, out=out):
                    for e in range(8):
                        rv = rb + (g * 8 + e)
                        for half in range(2):
                            kv = e * 2 + half
                            out[2 * g + kv // 8,
                                pl.ds((kv % 8) * 16, 16)] = (
                                plsc.load_gather(buf, [jv[half], rv]))
                pltpu.async_copy(
                    out,
                    tbl_out.at[pl.ds(pl.multiple_of(row0 + s * 32, 8),
                                     32), :],
                    semws[s])

        fire_read(jnp.int32(0), tins[0], semrs[0])

        def panel_pair(m, _):
            k0 = m * 2

            @pl.when(k0 + 1 <= KPW - 1)
            def _():
                fire_read(k0 + 1, tins[1], semrs[1])
            pltpu.make_async_copy(tbt_hbm.at[:, pl.ds(0, _W)],
                                  tins[0].at[:, pl.ds(0, _W)],
                                  semrs[0]).wait()
            compute_panel(k0, tins[0], m > 0)

            @pl.when(k0 + 2 <= KPW - 1)
            def _():
                fire_read(k0 + 2, tins[0], semrs[0])

            @pl.when(k0 + 1 <= KPW - 1)
            def _():
                pltpu.make_async_copy(tbt_hbm.at[:, pl.ds(0, _W)],
                                      tins[1].at[:, pl.ds(0, _W)],
                                      semrs[1]).wait()
                compute_panel(k0 + 1, tins[1], jnp.bool_(True))
            return 0

        lax.fori_loop(0, (KPW + 1) // 2, panel_pair, 0)
        if NEXTRA:
            @pl.when(wid == 0)
            def _():
                fire_read(jnp.int32(KPW), tins[0], semrs[0])
                pltpu.make_async_copy(tbt_hbm.at[:, pl.ds(0, _W)],
                                      tins[0].at[:, pl.ds(0, _W)],
                                      semrs[0]).wait()
                compute_panel(jnp.int32(KPW), tins[0], jnp.bool_(True))
        for b in range(4):
            drain_write(b)

        if VTAIL:
            @pl.when(wid == 0)
            def _():
                pltpu.sync_copy(tail_hbm, tailst)
                for el in range(VTAIL):
                    for half in range(2):
                        kv = el * 2 + half
                        tout0[kv // 8, pl.ds((kv % 8) * 16, 16)] = (
                            tailst[el, pl.ds(half * 16, 16)])
                pltpu.async_copy(
                    tout0.at[pl.ds(0, VTAIL * D // 128), :],
                    tbl_out.at[pl.ds(VFULL * D // 128, VTAIL * D // 128), :],
                    semw0)
                pltpu.make_async_copy(
                    tbl_out.at[pl.ds(0, VTAIL * D // 128), :],
                    tout0.at[pl.ds(0, VTAIL * D // 128), :], semw0).wait()

    return prep


def _build_run(F, B, L, V, D):
    NW = _NC * _NS
    NB = B // NW

    @functools.partial(
        pl.kernel,
        out_type=jax.ShapeDtypeStruct((B, F * D), jnp.float32),
        mesh=_mesh(),
        compiler_params=pltpu.CompilerParams(use_tc_tiling_on_sc=False),
        scratch_types=[
            pltpu.VMEM((L, NB), jnp.int32),
            pltpu.VMEM((L * NB, D), jnp.float32),
            pltpu.VMEM((NB, D), jnp.float32),
            pltpu.VMEM((F, L, _LANES), jnp.float32),
            pltpu.SemaphoreType.DMA,
        ],
    )
    def run(idx_hbm, table_hbm, pwe_hbm, out_hbm, idx_v, rows_v, out_v, pw_v,
            sem):
        wid = lax.axis_index("s") * _NC + lax.axis_index("c")
        pltpu.sync_copy(pwe_hbm, pw_v)

        def f_body(f, _):
            pltpu.sync_copy(idx_hbm.at[f, wid, pl.ds(0, L), :], idx_v)
            cps = [
                pltpu.async_copy(table_hbm.at[idx_v.at[l]],
                                 rows_v.at[pl.ds(l * NB, NB)], sem)
                for l in range(L)
            ]
            for c in cps:
                c.wait()
            wv = [pw_v[f, l, :] for l in range(L)]

            def bag(i, _):
                acc0 = jnp.zeros((_LANES,), jnp.float32)
                acc1 = jnp.zeros((_LANES,), jnp.float32)
                for l in range(L):
                    acc0 = acc0 + wv[l] * rows_v[l * NB + i, 0:16]
                    acc1 = acc1 + wv[l] * rows_v[l * NB + i, 16:32]
                out_v[i, 0:16] = acc0
                out_v[i, 16:32] = acc1
                return 0

            lax.fori_loop(0, NB, bag, 0)
            pltpu.sync_copy(out_v, out_hbm.at[pl.ds(wid * NB, NB),
                                              pl.ds(f * D, D)])
            return 0

        lax.fori_loop(0, F, f_body, 0)

    return run


def kernel(indices, table, pos_weight):
    F, B, L = indices.shape
    V, D = table.shape
    VFULL = (V // 128) * 128
    idx_t = jnp.transpose(indices.astype(jnp.int32), (0, 2, 1))
    pwe = jnp.broadcast_to(
        pos_weight.astype(jnp.float32)[:, :, None], (F, L, _LANES))
    tbt = jnp.transpose(table.astype(jnp.float32), (1, 0))
    tail = table.astype(jnp.float32)[VFULL:]
    tbl_dense, idx_dense = _build_prep(F, B, L, V, D)(tbt, tail, idx_t)
    tbl2 = tbl_dense.reshape(V, D)
    return _build_run(F, B, L, V, D)(idx_dense, tbl2, pwe)


# speedup vs baseline: 1.0132x; 1.0132x over previous
import functools

import jax
import jax.numpy as jnp
from jax import lax
from jax.experimental import pallas as pl
from jax.experimental.pallas import tpu as pltpu
from jax.experimental.pallas import tpu_sc as plsc

_NC = 2
_NS = 16
_LANES = 16


def _mesh():
    return plsc.VectorSubcoreMesh(
        core_axis_name="c", subcore_axis_name="s",
        num_cores=_NC, num_subcores=_NS)


_LPAD = 24
_W = 512


def _build_prep(F, B, L, V, D):
    NW = _NC * _NS
    NBLK = B // 128
    VFULL = (V // 128) * 128
    VTAIL = V - VFULL
    NPAN = VFULL // _W
    KPW = NPAN // NW
    NEXTRA = NPAN - KPW * NW
    assert NEXTRA <= 1 and VTAIL in (0, 64)
    OUTROWS = V * D // 128

    @functools.partial(
        pl.kernel,
        out_type=(
            jax.ShapeDtypeStruct((OUTROWS, 128), jnp.float32),
            jax.ShapeDtypeStruct((F, NBLK, _LPAD, 128), jnp.int32),
        ),
        mesh=_mesh(),
        compiler_params=pltpu.CompilerParams(use_tc_tiling_on_sc=True, needs_layout_passes=False),
        scratch_types=[
            pltpu.VMEM((_LPAD, 128), jnp.int32),
            pltpu.VMEM((_LPAD, 128), jnp.int32),
            pltpu.VMEM((D, _W + 1), jnp.float32),
            pltpu.VMEM((D, _W + 1), jnp.float32),
            pltpu.VMEM((32, 128), jnp.float32),
            pltpu.VMEM((32, 128), jnp.float32),
            pltpu.VMEM((VTAIL, D), jnp.float32),
            pltpu.SemaphoreType.DMA,
            pltpu.SemaphoreType.DMA,
            pltpu.SemaphoreType.DMA,
            pltpu.SemaphoreType.DMA,
            pltpu.SemaphoreType.DMA,
        ],
    )
    def prep(tbt_hbm, tail_hbm, idx_hbm, tbl_out, idx_out,
             ist0, ist1, tin0, tin1, tout0, tout1, tailst,
             sem_i, semr0, semr1, semw0, semw1):
        wid = lax.axis_index("s") * _NC + lax.axis_index("c")

        ists = [ist0, ist1]
        pending = [None, None]
        for f in range(F):
            stage = ists[f % 2]
            if pending[f % 2] is not None:
                pending[f % 2].wait()
                pending[f % 2] = None
            pltpu.sync_copy(
                idx_hbm.at[f, :, pl.ds(pl.multiple_of(wid * 128, 128), 128)],
                stage.at[pl.ds(0, L), :])
            pending[f % 2] = pltpu.async_copy(
                stage, idx_out.at[f, wid], [sem_i, semr0][f % 2])
        for cp in pending:
            if cp is not None:
                cp.wait()

        iota = lax.iota(jnp.int32, _LANES)
        jv = [iota, iota + _LANES]
        tins = [tin0, tin1]
        semrs = [semr0, semr1]
        touts = [tout0, tout1]
        semws = [semw0, semw1]
        p_extra = jnp.int32(NPAN - 1)

        def p_of(k):
            base = wid * KPW + k
            if NEXTRA:
                return jnp.where(k == KPW, p_extra, base)
            return base

        def fire_read(k, buf, sem):
            c0 = pl.multiple_of(p_of(k) * _W, _W)
            return pltpu.async_copy(tbt_hbm.at[:, pl.ds(c0, _W)],
                                buf.at[:, pl.ds(0, _W)], sem)

        def drain_write(b):
            pltpu.make_async_copy(tbl_out.at[pl.ds(0, 32), :], touts[b],
                                  semws[b]).wait()

        def compute_panel(k, buf, not_first):
            p = p_of(k)
            row0 = p * 128

            def sub(t, _):
                for b in range(2):
                    s = t * 2 + b

                    @pl.when(jnp.logical_or(not_first, t > 0))
                    def _():
                        drain_write(b)
                    rb = jnp.zeros((_LANES,), jnp.int32) + s * 128
                    out = touts[b]

                    @plsc.parallel_loop(0, 16, step=1, unroll=4)
                    def grp(g):
                        for e in range(8):
                            rv = rb + (g * 8 + e)
                            for half in range(2):
                                kv = e * 2 + half
                                out[2 * g + kv // 8,
                                    pl.ds((kv % 8) * 16, 16)] = (
                                    plsc.load_gather(buf, [jv[half], rv]))
                    pltpu.async_copy(
                        out,
                        tbl_out.at[pl.ds(pl.multiple_of(row0 + s * 32, 8),
                                         32), :],
                        semws[b])
                return 0

            lax.fori_loop(0, 2, sub, 0)

        fire_read(jnp.int32(0), tins[0], semrs[0])

        def panel_pair(m, _):
            k0 = m * 2

            @pl.when(k0 + 1 <= KPW - 1)
            def _():
                fire_read(k0 + 1, tins[1], semrs[1])
            pltpu.make_async_copy(tbt_hbm.at[:, pl.ds(0, _W)],
                                  tins[0].at[:, pl.ds(0, _W)],
                                  semrs[0]).wait()
            compute_panel(k0, tins[0], m > 0)

            @pl.when(k0 + 2 <= KPW - 1)
            def _():
                fire_read(k0 + 2, tins[0], semrs[0])

            @pl.when(k0 + 1 <= KPW - 1)
            def _():
                pltpu.make_async_copy(tbt_hbm.at[:, pl.ds(0, _W)],
                                      tins[1].at[:, pl.ds(0, _W)],
                                      semrs[1]).wait()
                compute_panel(k0 + 1, tins[1], jnp.bool_(True))
            return 0

        lax.fori_loop(0, (KPW + 1) // 2, panel_pair, 0)
        if NEXTRA:
            @pl.when(wid == 0)
            def _():
                fire_read(jnp.int32(KPW), tins[0], semrs[0])
                pltpu.make_async_copy(tbt_hbm.at[:, pl.ds(0, _W)],
                                      tins[0].at[:, pl.ds(0, _W)],
                                      semrs[0]).wait()
                compute_panel(jnp.int32(KPW), tins[0], jnp.bool_(True))
        for b in range(2):
            drain_write(b)

        if VTAIL:
            @pl.when(wid == 0)
            def _():
                pltpu.sync_copy(tail_hbm, tailst)
                for el in range(VTAIL):
                    for half in range(2):
                        kv = el * 2 + half
                        tout0[kv // 8, pl.ds((kv % 8) * 16, 16)] = (
                            tailst[el, pl.ds(half * 16, 16)])
                pltpu.async_copy(
                    tout0.at[pl.ds(0, VTAIL * D // 128), :],
                    tbl_out.at[pl.ds(VFULL * D // 128, VTAIL * D // 128), :],
                    semw0)
                pltpu.make_async_copy(
                    tbl_out.at[pl.ds(0, VTAIL * D // 128), :],
                    tout0.at[pl.ds(0, VTAIL * D // 128), :], semw0).wait()

    return prep


def _build_run(F, B, L, V, D):
    NW = _NC * _NS
    NB = B // NW

    @functools.partial(
        pl.kernel,
        out_type=jax.ShapeDtypeStruct((B, F * D), jnp.float32),
        mesh=_mesh(),
        compiler_params=pltpu.CompilerParams(use_tc_tiling_on_sc=False),
        scratch_types=[
            pltpu.VMEM((L, NB), jnp.int32),
            pltpu.VMEM((L * NB, D), jnp.float32),
            pltpu.VMEM((NB, D), jnp.float32),
            pltpu.VMEM((F, L, _LANES), jnp.float32),
            pltpu.SemaphoreType.DMA,
        ],
    )
    def run(idx_hbm, table_hbm, pwe_hbm, out_hbm, idx_v, rows_v, out_v, pw_v,
            sem):
        wid = lax.axis_index("s") * _NC + lax.axis_index("c")
        pltpu.sync_copy(pwe_hbm, pw_v)

        def f_body(f, _):
            pltpu.sync_copy(idx_hbm.at[f, wid, pl.ds(0, L), :], idx_v)
            cps = [
                pltpu.async_copy(table_hbm.at[idx_v.at[l]],
                                 rows_v.at[pl.ds(l * NB, NB)], sem)
                for l in range(L)
            ]
            for c in cps:
                c.wait()
            wv = [pw_v[f, l, :] for l in range(L)]

            def bag(i, _):
                acc0 = jnp.zeros((_LANES,), jnp.float32)
                acc1 = jnp.zeros((_LANES,), jnp.float32)
                for l in range(L):
                    acc0 = acc0 + wv[l] * rows_v[l * NB + i, 0:16]
                    acc1 = acc1 + wv[l] * rows_v[l * NB + i, 16:32]
                out_v[i, 0:16] = acc0
                out_v[i, 16:32] = acc1
                return 0

            lax.fori_loop(0, NB, bag, 0)
            pltpu.sync_copy(out_v, out_hbm.at[pl.ds(wid * NB, NB),
                                              pl.ds(f * D, D)])
            return 0

        lax.fori_loop(0, F, f_body, 0)

    return run


def kernel(indices, table, pos_weight):
    F, B, L = indices.shape
    V, D = table.shape
    VFULL = (V // 128) * 128
    idx_t = jnp.transpose(indices.astype(jnp.int32), (0, 2, 1))
    pwe = jnp.broadcast_to(
        pos_weight.astype(jnp.float32)[:, :, None], (F, L, _LANES))
    tbt = jnp.transpose(table.astype(jnp.float32), (1, 0))
    tail = table.astype(jnp.float32)[VFULL:]
    tbl_dense, idx_dense = _build_prep(F, B, L, V, D)(tbt, tail, idx_t)
    tbl2 = tbl_dense.reshape(V, D)
    return _build_run(F, B, L, V, D)(idx_dense, tbl2, pwe)
